# baseline (speedup 1.0000x reference)
import functools

import jax
import jax.numpy as jnp
from jax import lax
from jax.experimental import pallas as pl
from jax.experimental.pallas import tpu as pltpu
from jax.experimental.pallas import tpu_sc as plsc

N_NODES = 10000
D_IN = 128
D_HID = 128
D_HALF = D_HID // 2
D_OUT = 64
N_EDGES = 320000

NC = 2
NS = 16
LANES = 16
CHUNK = 128
N_PAD = 10240
ROWS_PER_TILE = N_PAD // NS
GROUP = 4
NG = -(-N_EDGES // (NS * GROUP * CHUNK))
E_PAD = NS * NG * GROUP * CHUNK
C_DEG = -(-N_EDGES // (NC * NS * CHUNK))
E_PAD_DEG = NC * NS * C_DEG * CHUNK

_mesh = plsc.VectorSubcoreMesh(
    core_axis_name="c", subcore_axis_name="s", num_cores=NC, num_subcores=NS
)


@functools.partial(
    pl.kernel,
    out_type=jax.ShapeDtypeStruct((NC, N_PAD), jnp.float32),
    mesh=_mesh,
    scratch_types=[
        pltpu.VMEM((C_DEG, CHUNK), jnp.int32),
        pltpu.VMEM((ROWS_PER_TILE,), jnp.float32),
        pltpu.VMEM_SHARED((N_PAD,), jnp.float32),
    ],
)
def _deg_call(dst_hbm, deg_out, dst_v, ones_v, deg_s):
    c = lax.axis_index("c")
    s = lax.axis_index("s")
    row0 = s * ROWS_PER_TILE

    def fill_ones(k, carry):
        ones_v[pl.ds(k * LANES, LANES)] = jnp.ones((LANES,), jnp.float32)
        return carry

    lax.fori_loop(0, ROWS_PER_TILE // LANES, fill_ones, 0)
    pltpu.sync_copy(ones_v, deg_s.at[pl.ds(row0, ROWS_PER_TILE)])
    pltpu.sync_copy(dst_hbm.at[c, s], dst_v)
    plsc.subcore_barrier()

    def body(j, carry):
        pltpu.sync_copy(
            ones_v.at[pl.ds(0, CHUNK)], deg_s.at[dst_v.at[j]], add=True
        )
        return carry

    lax.fori_loop(0, C_DEG, body, 0)
    plsc.subcore_barrier()
    pltpu.sync_copy(
        deg_s.at[pl.ds(row0, ROWS_PER_TILE)],
        deg_out.at[c, pl.ds(row0, ROWS_PER_TILE)],
    )


@functools.partial(
    pl.kernel,
    out_type=jax.ShapeDtypeStruct((NC, N_PAD, D_HALF), jnp.float32),
    mesh=_mesh,
    scratch_types=[
        pltpu.VMEM((NG, GROUP * CHUNK), jnp.int32),
        pltpu.VMEM((2, GROUP * CHUNK), jnp.int32),
        pltpu.VMEM((GROUP * CHUNK, D_HALF), jnp.float32),
        pltpu.VMEM((GROUP * CHUNK, D_HALF), jnp.float32),
        pltpu.VMEM_SHARED((N_PAD, D_HALF), jnp.float32),
        pltpu.SemaphoreType.DMA,
        pltpu.SemaphoreType.DMA,
        pltpu.SemaphoreType.DMA,
        pltpu.SemaphoreType.DMA,
    ],
    compiler_params=pltpu.CompilerParams(use_tc_tiling_on_sc=False),
)
def _mp_call(g_hbm, edge_hbm, acc_out,
             src_v, dst_v, rows_a, rows_b, acc_s, sem_a, sem_b, sem_da, sem_db):
    c = lax.axis_index("c")
    s = lax.axis_index("s")
    row0 = s * ROWS_PER_TILE
    g_c = g_hbm.at[c]
    src_hbm = edge_hbm.at[0]
    dst_hbm = edge_hbm.at[1]

    pltpu.sync_copy(
        g_c.at[pl.ds(row0, ROWS_PER_TILE)],
        acc_s.at[pl.ds(row0, ROWS_PER_TILE)],
    )
    pltpu.sync_copy(src_hbm.at[s], src_v)
    plsc.subcore_barrier()

    pltpu.async_copy(g_c.at[src_v.at[0]], rows_a, sem_a)
    pltpu.async_copy(dst_hbm.at[s, 0], dst_v.at[0], sem_da)
    pltpu.async_copy(g_c.at[src_v.at[1]], rows_b, sem_b)
    pltpu.async_copy(dst_hbm.at[s, 1], dst_v.at[1], sem_db)

    def body(t, carry):
        j = 2 * t
        pltpu.make_async_copy(g_c.at[src_v.at[j]], rows_a, sem_a).wait()
        pltpu.make_async_copy(dst_hbm.at[s, j], dst_v.at[0], sem_da).wait()
        pltpu.sync_copy(rows_a, acc_s.at[dst_v.at[0]], add=True)

        @pl.when(j + 2 < NG)
        def _():
            pltpu.async_copy(g_c.at[src_v.at[j + 2]], rows_a, sem_a)
            pltpu.async_copy(dst_hbm.at[s, j + 2], dst_v.at[0], sem_da)

        @pl.when(j + 1 < NG)
        def _():
            pltpu.make_async_copy(
                g_c.at[src_v.at[j + 1]], rows_b, sem_b
            ).wait()
            pltpu.make_async_copy(
                dst_hbm.at[s, j + 1], dst_v.at[1], sem_db
            ).wait()
            pltpu.sync_copy(rows_b, acc_s.at[dst_v.at[1]], add=True)

        @pl.when(j + 3 < NG)
        def _():
            pltpu.async_copy(g_c.at[src_v.at[j + 3]], rows_b, sem_b)
            pltpu.async_copy(dst_hbm.at[s, j + 3], dst_v.at[1], sem_db)

        return carry

    lax.fori_loop(0, (NG + 1) // 2, body, 0)
    plsc.subcore_barrier()
    pltpu.sync_copy(
        acc_s.at[pl.ds(row0, ROWS_PER_TILE)],
        acc_out.at[c, pl.ds(row0, ROWS_PER_TILE)],
    )


_BR = 1280


def _dense0_body(x_ref, w_ref, h_ref):
    h_ref[...] = jnp.dot(
        x_ref[...], w_ref[...], preferred_element_type=jnp.float32
    )


def _dense0_call(x_pad, w1):
    return pl.pallas_call(
        _dense0_body,
        grid=(N_PAD // _BR,),
        in_specs=[
            pl.BlockSpec((_BR, D_IN), lambda i: (i, 0)),
            pl.BlockSpec((D_IN, D_HID), lambda i: (0, 0)),
        ],
        out_specs=pl.BlockSpec((_BR, D_HID), lambda i: (i, 0)),
        out_shape=jax.ShapeDtypeStruct((N_PAD, D_HID), jnp.float32),
    )(x_pad, w1)


def _dense1_body(h_ref, d0_ref, d1_ref, g_ref, dinv_ref):
    i = pl.program_id(0)
    deg = d0_ref[...] + d1_ref[...] - 1.0
    rows = i * _BR + lax.broadcasted_iota(jnp.int32, (_BR, 1), 0)
    dinv = jnp.where(rows < N_NODES, lax.rsqrt(deg), 0.0)
    dinv_ref[...] = dinv
    res = dinv * h_ref[...]
    g_ref[0] = res[:, :D_HALF]
    g_ref[1] = res[:, D_HALF:]


def _dense1_call(h1, deg0, deg1):
    return pl.pallas_call(
        _dense1_body,
        grid=(N_PAD // _BR,),
        in_specs=[
            pl.BlockSpec((_BR, D_HID), lambda i: (i, 0)),
            pl.BlockSpec((_BR, 1), lambda i: (i, 0)),
            pl.BlockSpec((_BR, 1), lambda i: (i, 0)),
        ],
        out_specs=[
            pl.BlockSpec((NC, _BR, D_HALF), lambda i: (0, i, 0)),
            pl.BlockSpec((_BR, 1), lambda i: (i, 0)),
        ],
        out_shape=[
            jax.ShapeDtypeStruct((NC, N_PAD, D_HALF), jnp.float32),
            jax.ShapeDtypeStruct((N_PAD, 1), jnp.float32),
        ],
    )(h1, deg0, deg1)


def _dense2_body(a_ref, dinv_ref, b_ref, w_ref, g_ref):
    dinv = dinv_ref[...]
    acc = jnp.concatenate([a_ref[0], a_ref[1]], axis=1)
    m = jax.nn.relu(dinv * acc + b_ref[...])
    res = dinv * jnp.dot(m, w_ref[...], preferred_element_type=jnp.float32)
    g_ref[0] = res[:, :D_HALF]
    g_ref[1] = res[:, D_HALF:]


def _dense2_call(acc, dinv, b1, w2):
    return pl.pallas_call(
        _dense2_body,
        grid=(N_PAD // _BR,),
        in_specs=[
            pl.BlockSpec((NC, _BR, D_HALF), lambda i: (0, i, 0)),
            pl.BlockSpec((_BR, 1), lambda i: (i, 0)),
            pl.BlockSpec((1, D_HID), lambda i: (0, 0)),
            pl.BlockSpec((D_HID, D_HID), lambda i: (0, 0)),
        ],
        out_specs=pl.BlockSpec((NC, _BR, D_HALF), lambda i: (0, i, 0)),
        out_shape=jax.ShapeDtypeStruct((NC, N_PAD, D_HALF), jnp.float32),
    )(acc, dinv, b1, w2)


def _dense3_body(a_ref, dinv_ref, b_ref, w_ref, bfc_ref, o_ref):
    acc = jnp.concatenate([a_ref[0], a_ref[1]], axis=1)
    m = jax.nn.relu(dinv_ref[...] * acc + b_ref[...])
    o_ref[...] = (
        jnp.dot(m, w_ref[...], preferred_element_type=jnp.float32)
        + bfc_ref[...]
    )


def _dense3_call(acc, dinv, b2, wfc, bfc):
    return pl.pallas_call(
        _dense3_body,
        grid=(N_PAD // _BR,),
        in_specs=[
            pl.BlockSpec((NC, _BR, D_HALF), lambda i: (0, i, 0)),
            pl.BlockSpec((_BR, 1), lambda i: (i, 0)),
            pl.BlockSpec((1, D_HID), lambda i: (0, 0)),
            pl.BlockSpec((D_HID, D_OUT), lambda i: (0, 0)),
            pl.BlockSpec((1, D_OUT), lambda i: (0, 0)),
        ],
        out_specs=pl.BlockSpec((_BR, D_OUT), lambda i: (i, 0)),
        out_shape=jax.ShapeDtypeStruct((N_PAD, D_OUT), jnp.float32),
    )(acc, dinv, b2, wfc, bfc)


def kernel(x, edge_index, W1, b1, W2, b2, Wfc, bfc):
    x_pad = jnp.concatenate(
        [x, jnp.zeros((N_PAD - N_NODES, D_IN), jnp.float32)], axis=0
    )

    src32 = edge_index[0].astype(jnp.int32)
    dst32 = edge_index[1].astype(jnp.int32)
    padf = N_NODES + jnp.arange(E_PAD - N_EDGES, dtype=jnp.int32) % (
        N_PAD - N_NODES
    )
    edge_r = jnp.stack(
        [
            jnp.concatenate([src32, padf]).reshape(NS, NG, GROUP * CHUNK),
            jnp.concatenate([dst32, padf]).reshape(NS, NG, GROUP * CHUNK),
        ]
    )
    padd = jnp.full((E_PAD_DEG - N_EDGES,), N_NODES, jnp.int32)
    dst_deg = jnp.concatenate([dst32, padd]).reshape(NC, NS, C_DEG, CHUNK)

    h1 = _dense0_call(x_pad, W1)
    degp = _deg_call(dst_deg)
    deg0 = degp[0].reshape(N_PAD, 1)
    deg1 = degp[1].reshape(N_PAD, 1)

    g1, dinv = _dense1_call(h1, deg0, deg1)

    acc1 = _mp_call(g1, edge_r)
    g2 = _dense2_call(acc1, dinv, b1.reshape(1, D_HID), W2)

    acc2 = _mp_call(g2, edge_r)
    out = _dense3_call(
        acc2, dinv, b2.reshape(1, D_HID), Wfc, bfc.reshape(1, D_OUT)
    )
    return out[:N_NODES]


# speedup vs baseline: 1.0057x; 1.0057x over previous
import functools

import jax
import jax.numpy as jnp
from jax import lax
from jax.experimental import pallas as pl
from jax.experimental.pallas import tpu as pltpu
from jax.experimental.pallas import tpu_sc as plsc

N_NODES = 10000
D_IN = 128
D_HID = 128
D_HALF = D_HID // 2
D_OUT = 64
N_EDGES = 320000

NC = 2
NS = 16
LANES = 16
CHUNK = 128
N_PAD = 10240
ROWS_PER_TILE = N_PAD // NS
GROUP = 4
NG = -(-N_EDGES // (NS * GROUP * CHUNK))
E_PAD = NS * NG * GROUP * CHUNK
C_DEG = -(-N_EDGES // (NC * NS * CHUNK))
E_PAD_DEG = NC * NS * C_DEG * CHUNK

_mesh = plsc.VectorSubcoreMesh(
    core_axis_name="c", subcore_axis_name="s", num_cores=NC, num_subcores=NS
)


@functools.partial(
    pl.kernel,
    out_type=jax.ShapeDtypeStruct((NC, N_PAD), jnp.float32),
    mesh=_mesh,
    scratch_types=[
        pltpu.VMEM((C_DEG, CHUNK), jnp.int32),
        pltpu.VMEM((ROWS_PER_TILE,), jnp.float32),
        pltpu.VMEM_SHARED((N_PAD,), jnp.float32),
    ],
)
def _deg_call(dst_hbm, deg_out, dst_v, ones_v, deg_s):
    c = lax.axis_index("c")
    s = lax.axis_index("s")
    row0 = s * ROWS_PER_TILE

    def fill_ones(k, carry):
        ones_v[pl.ds(k * LANES, LANES)] = jnp.ones((LANES,), jnp.float32)
        return carry

    lax.fori_loop(0, ROWS_PER_TILE // LANES, fill_ones, 0)
    pltpu.sync_copy(ones_v, deg_s.at[pl.ds(row0, ROWS_PER_TILE)])
    pltpu.sync_copy(dst_hbm.at[c, s], dst_v)
    plsc.subcore_barrier()

    def body(j, carry):
        pltpu.sync_copy(
            ones_v.at[pl.ds(0, CHUNK)], deg_s.at[dst_v.at[j]], add=True
        )
        return carry

    lax.fori_loop(0, C_DEG, body, 0)
    plsc.subcore_barrier()
    pltpu.sync_copy(
        deg_s.at[pl.ds(row0, ROWS_PER_TILE)],
        deg_out.at[c, pl.ds(row0, ROWS_PER_TILE)],
    )


@functools.partial(
    pl.kernel,
    out_type=jax.ShapeDtypeStruct((NC, N_PAD, D_HALF), jnp.float32),
    mesh=_mesh,
    scratch_types=[
        pltpu.VMEM((NG, GROUP * CHUNK), jnp.int32),
        pltpu.VMEM((2, GROUP * CHUNK), jnp.int32),
        pltpu.VMEM((GROUP * CHUNK, D_HALF), jnp.float32),
        pltpu.VMEM((GROUP * CHUNK, D_HALF), jnp.float32),
        pltpu.VMEM_SHARED((N_PAD, D_HALF), jnp.float32),
        pltpu.SemaphoreType.DMA,
        pltpu.SemaphoreType.DMA,
        pltpu.SemaphoreType.DMA,
        pltpu.SemaphoreType.DMA,
    ],
    compiler_params=pltpu.CompilerParams(use_tc_tiling_on_sc=False),
)
def _mp_call(g_hbm, edge_hbm, acc_out,
             src_v, dst_v, rows_a, rows_b, acc_s, sem_a, sem_b, sem_da, sem_db):
    c = lax.axis_index("c")
    s = lax.axis_index("s")
    row0 = s * ROWS_PER_TILE
    g_c = g_hbm.at[c]
    src_hbm = edge_hbm.at[0]
    dst_hbm = edge_hbm.at[1]

    pltpu.sync_copy(
        g_c.at[pl.ds(row0, ROWS_PER_TILE)],
        acc_s.at[pl.ds(row0, ROWS_PER_TILE)],
    )
    pltpu.sync_copy(src_hbm.at[s], src_v)
    plsc.subcore_barrier()

    pltpu.async_copy(g_c.at[src_v.at[0]], rows_a, sem_a)
    pltpu.async_copy(dst_hbm.at[s, 0], dst_v.at[0], sem_da)
    pltpu.async_copy(g_c.at[src_v.at[1]], rows_b, sem_b)
    pltpu.async_copy(dst_hbm.at[s, 1], dst_v.at[1], sem_db)

    def body(t, carry):
        j = 2 * t
        pltpu.make_async_copy(g_c.at[src_v.at[j]], rows_a, sem_a).wait()
        pltpu.make_async_copy(dst_hbm.at[s, j], dst_v.at[0], sem_da).wait()
        pltpu.sync_copy(rows_a, acc_s.at[dst_v.at[0]], add=True)

        @pl.when(j + 2 < NG)
        def _():
            pltpu.async_copy(g_c.at[src_v.at[j + 2]], rows_a, sem_a)
            pltpu.async_copy(dst_hbm.at[s, j + 2], dst_v.at[0], sem_da)

        @pl.when(j + 1 < NG)
        def _():
            pltpu.make_async_copy(
                g_c.at[src_v.at[j + 1]], rows_b, sem_b
            ).wait()
            pltpu.make_async_copy(
                dst_hbm.at[s, j + 1], dst_v.at[1], sem_db
            ).wait()
            pltpu.sync_copy(rows_b, acc_s.at[dst_v.at[1]], add=True)

        @pl.when(j + 3 < NG)
        def _():
            pltpu.async_copy(g_c.at[src_v.at[j + 3]], rows_b, sem_b)
            pltpu.async_copy(dst_hbm.at[s, j + 3], dst_v.at[1], sem_db)

        return carry

    lax.fori_loop(0, (NG + 1) // 2, body, 0)
    plsc.subcore_barrier()
    pltpu.sync_copy(
        acc_s.at[pl.ds(row0, ROWS_PER_TILE)],
        acc_out.at[c, pl.ds(row0, ROWS_PER_TILE)],
    )


_BR = 1280


def _dense1_body(x_ref, w_ref, d0_ref, d1_ref, g_ref, dinv_ref):
    i = pl.program_id(0)
    deg = d0_ref[...] + d1_ref[...] - 1.0
    rows = i * _BR + lax.broadcasted_iota(jnp.int32, (_BR, 1), 0)
    dinv = jnp.where(rows < N_NODES, lax.rsqrt(deg), 0.0)
    dinv_ref[...] = dinv
    res = dinv * jnp.dot(
        x_ref[...], w_ref[...], preferred_element_type=jnp.float32
    )
    g_ref[0] = res[:, :D_HALF]
    g_ref[1] = res[:, D_HALF:]


def _dense1_call(x_pad, w1, deg0, deg1):
    return pl.pallas_call(
        _dense1_body,
        grid=(N_PAD // _BR,),
        in_specs=[
            pl.BlockSpec((_BR, D_IN), lambda i: (i, 0)),
            pl.BlockSpec((D_IN, D_HID), lambda i: (0, 0)),
            pl.BlockSpec((_BR, 1), lambda i: (i, 0)),
            pl.BlockSpec((_BR, 1), lambda i: (i, 0)),
        ],
        out_specs=[
            pl.BlockSpec((NC, _BR, D_HALF), lambda i: (0, i, 0)),
            pl.BlockSpec((_BR, 1), lambda i: (i, 0)),
        ],
        out_shape=[
            jax.ShapeDtypeStruct((NC, N_PAD, D_HALF), jnp.float32),
            jax.ShapeDtypeStruct((N_PAD, 1), jnp.float32),
        ],
    )(x_pad, w1, deg0, deg1)


def _dense2_body(a_ref, dinv_ref, b_ref, w_ref, g_ref):
    dinv = dinv_ref[...]
    acc = jnp.concatenate([a_ref[0], a_ref[1]], axis=1)
    m = jax.nn.relu(dinv * acc + b_ref[...])
    res = dinv * jnp.dot(m, w_ref[...], preferred_element_type=jnp.float32)
    g_ref[0] = res[:, :D_HALF]
    g_ref[1] = res[:, D_HALF:]


def _dense2_call(acc, dinv, b1, w2):
    return pl.pallas_call(
        _dense2_body,
        grid=(N_PAD // _BR,),
        in_specs=[
            pl.BlockSpec((NC, _BR, D_HALF), lambda i: (0, i, 0)),
            pl.BlockSpec((_BR, 1), lambda i: (i, 0)),
            pl.BlockSpec((1, D_HID), lambda i: (0, 0)),
            pl.BlockSpec((D_HID, D_HID), lambda i: (0, 0)),
        ],
        out_specs=pl.BlockSpec((NC, _BR, D_HALF), lambda i: (0, i, 0)),
        out_shape=jax.ShapeDtypeStruct((NC, N_PAD, D_HALF), jnp.float32),
    )(acc, dinv, b1, w2)


def _dense3_body(a_ref, dinv_ref, b_ref, w_ref, bfc_ref, o_ref):
    acc = jnp.concatenate([a_ref[0], a_ref[1]], axis=1)
    m = jax.nn.relu(dinv_ref[...] * acc + b_ref[...])
    o_ref[...] = (
        jnp.dot(m, w_ref[...], preferred_element_type=jnp.float32)
        + bfc_ref[...]
    )


def _dense3_call(acc, dinv, b2, wfc, bfc):
    return pl.pallas_call(
        _dense3_body,
        grid=(N_PAD // _BR,),
        in_specs=[
            pl.BlockSpec((NC, _BR, D_HALF), lambda i: (0, i, 0)),
            pl.BlockSpec((_BR, 1), lambda i: (i, 0)),
            pl.BlockSpec((1, D_HID), lambda i: (0, 0)),
            pl.BlockSpec((D_HID, D_OUT), lambda i: (0, 0)),
            pl.BlockSpec((1, D_OUT), lambda i: (0, 0)),
        ],
        out_specs=pl.BlockSpec((_BR, D_OUT), lambda i: (i, 0)),
        out_shape=jax.ShapeDtypeStruct((N_PAD, D_OUT), jnp.float32),
    )(acc, dinv, b2, wfc, bfc)


def kernel(x, edge_index, W1, b1, W2, b2, Wfc, bfc):
    x_pad = jnp.concatenate(
        [x, jnp.zeros((N_PAD - N_NODES, D_IN), jnp.float32)], axis=0
    )

    src32 = edge_index[0].astype(jnp.int32)
    dst32 = edge_index[1].astype(jnp.int32)
    padf = N_NODES + jnp.arange(E_PAD - N_EDGES, dtype=jnp.int32) % (
        N_PAD - N_NODES
    )
    edge_r = jnp.stack(
        [
            jnp.concatenate([src32, padf]).reshape(NS, NG, GROUP * CHUNK),
            jnp.concatenate([dst32, padf]).reshape(NS, NG, GROUP * CHUNK),
        ]
    )
    padd = jnp.full((E_PAD_DEG - N_EDGES,), N_NODES, jnp.int32)
    dst_deg = jnp.concatenate([dst32, padd]).reshape(NC, NS, C_DEG, CHUNK)

    degp = _deg_call(dst_deg)
    deg0 = degp[0].reshape(N_PAD, 1)
    deg1 = degp[1].reshape(N_PAD, 1)

    g1, dinv = _dense1_call(x_pad, W1, deg0, deg1)

    acc1 = _mp_call(g1, edge_r)
    g2 = _dense2_call(acc1, dinv, b1.reshape(1, D_HID), W2)

    acc2 = _mp_call(g2, edge_r)
    out = _dense3_call(
        acc2, dinv, b2.reshape(1, D_HID), Wfc, bfc.reshape(1, D_OUT)
    )
    return out[:N_NODES]
